# baseline (speedup 1.0000x reference)
import functools

import jax
import jax.numpy as jnp
from jax import lax
from jax.experimental import pallas as pl
from jax.experimental.pallas import tpu as pltpu
from jax.experimental.pallas import tpu_sc as plsc

_BATCH = 4096
_STATE = 32
_DICT = 8192
_EMB = 64

_NC = 2
_NS = 16
_NW = _NC * _NS

_ROWS = _BATCH * _STATE
_RPW = _ROWS // _NW
_CHUNK = 128
_NCHUNK = _RPW // _CHUNK
_K = 4
_NG = _NCHUNK // _K
_NBUF = 2 * _K


def _gather_body(table_hbm, idx_hbm, out_hbm, idx_v, bufs, sg0, sg1, ss0, ss1):
    wid = lax.axis_index("s") * _NC + lax.axis_index("c")
    base = wid * _RPW
    pltpu.sync_copy(idx_hbm.at[pl.ds(base, _RPW)], idx_v)

    sems_g = (sg0, sg1)
    sems_s = (ss0, ss1)

    def fire_gathers(g):
        descs = []
        for j in range(_K):
            c = g * _K + j
            b = (g % 2) * _K + j
            d = pltpu.make_async_copy(
                table_hbm.at[idx_v.at[pl.ds(c * _CHUNK, _CHUNK)]],
                bufs.at[b], sems_g[g % 2])
            d.start()
            descs.append(d)
        return descs

    def fire_stores(g):
        descs = []
        for j in range(_K):
            c = g * _K + j
            b = (g % 2) * _K + j
            d = pltpu.make_async_copy(
                bufs.at[b],
                out_hbm.at[pl.ds(base + c * _CHUNK, _CHUNK)], sems_s[g % 2])
            d.start()
            descs.append(d)
        return descs

    store_descs = {}
    gather_descs = {0: fire_gathers(0)}
    for g in range(_NG):
        if g + 1 < _NG:
            if g - 1 >= 0:
                for d in store_descs.pop(g - 1):
                    d.wait()
            gather_descs[g + 1] = fire_gathers(g + 1)
        for d in gather_descs.pop(g):
            d.wait()
        store_descs[g] = fire_stores(g)
    for g in sorted(store_descs):
        for d in store_descs[g]:
            d.wait()


@functools.partial(
    pl.kernel,
    mesh=plsc.VectorSubcoreMesh(core_axis_name="c", subcore_axis_name="s"),
    out_type=jax.ShapeDtypeStruct((_ROWS, _EMB), jnp.float32),
    compiler_params=pltpu.CompilerParams(use_tc_tiling_on_sc=False),
    scratch_types=[
        pltpu.VMEM((_RPW,), jnp.int32),
        pltpu.VMEM((_NBUF, _CHUNK, _EMB), jnp.float32),
        pltpu.SemaphoreType.DMA,
        pltpu.SemaphoreType.DMA,
        pltpu.SemaphoreType.DMA,
        pltpu.SemaphoreType.DMA,
    ],
)
def _gather(table_hbm, idx_hbm, out_hbm, idx_v, bufs, sg0, sg1, ss0, ss1):
    _gather_body(table_hbm, idx_hbm, out_hbm, idx_v, bufs, sg0, sg1, ss0, ss1)


def kernel(indices, embedding, ema_cluster_size, ema_w):
    del ema_cluster_size, ema_w
    table = embedding.reshape(_STATE * _DICT, _EMB)
    s_off = jnp.arange(_STATE, dtype=jnp.int32)[None, :] * _DICT
    flat_idx = (indices + s_off).reshape(_ROWS)
    out = _gather(table, flat_idx)
    return out.reshape(_BATCH, _STATE, _EMB)
